# baseline (speedup 1.0000x reference)
import functools

import jax
import jax.numpy as jnp
from jax import lax
from jax.experimental import pallas as pl
from jax.experimental.pallas import tpu as pltpu
from jax.experimental.pallas import tpu_sc as plsc

EMB = 64
NC = 2
NS = 16
NW = NC * NS
CHUNK = 128


@functools.lru_cache(maxsize=None)
def _make_gather(B: int, V: int):
    b_per_w = B // NW
    n_chunks = b_per_w // CHUNK
    mesh = plsc.VectorSubcoreMesh(core_axis_name="c", subcore_axis_name="s")

    @functools.partial(
        pl.kernel,
        mesh=mesh,
        out_type=jax.ShapeDtypeStruct((B, EMB), jnp.float32),
        scratch_types=[
            pltpu.VMEM((n_chunks, CHUNK), jnp.int32),
            pltpu.VMEM((CHUNK, EMB), jnp.float32),
            pltpu.SemaphoreType.DMA,
        ],
        compiler_params=pltpu.CompilerParams(use_tc_tiling_on_sc=False),
    )
    def k(idx_hbm, table_hbm, out_hbm, idx_v, rows_v, sem):
        wid = lax.axis_index("s") * NC + lax.axis_index("c")
        pltpu.sync_copy(idx_hbm.at[wid], idx_v)
        base = wid * b_per_w

        def step(j, carry):
            pltpu.async_copy(table_hbm.at[idx_v.at[j]], rows_v, sem).wait()
            pltpu.sync_copy(rows_v, out_hbm.at[pl.ds(base + j * CHUNK, CHUNK)])
            return carry

        lax.fori_loop(0, n_chunks, step, 0)

    return k


def kernel(y, table):
    batch, n_fields = y.shape
    B = batch * n_fields
    idx = y.astype(jnp.int32).reshape(NW, B // NW // CHUNK, CHUNK)
    out = _make_gather(B, table.shape[0])(idx, table)
    return out.reshape(batch, n_fields, EMB)


# speedup vs baseline: 1.0794x; 1.0794x over previous
import functools

import jax
import jax.numpy as jnp
from jax import lax
from jax.experimental import pallas as pl
from jax.experimental.pallas import tpu as pltpu
from jax.experimental.pallas import tpu_sc as plsc

EMB = 64
NC = 2
NS = 16
NW = NC * NS
CHUNK = 128


NACC = 4
GROUP_ROWS = NACC * CHUNK


@functools.lru_cache(maxsize=None)
def _make_gather(B: int, V: int):
    b_per_w = B // NW
    n_chunks = b_per_w // CHUNK
    n_groups = n_chunks // NACC
    assert n_chunks % NACC == 0 and n_groups % 2 == 0
    mesh = plsc.VectorSubcoreMesh(core_axis_name="c", subcore_axis_name="s")

    @functools.partial(
        pl.kernel,
        mesh=mesh,
        out_type=jax.ShapeDtypeStruct((B, EMB), jnp.float32),
        scratch_types=[
            pltpu.VMEM((n_chunks, CHUNK), jnp.int32),
            pltpu.VMEM((GROUP_ROWS, EMB), jnp.float32),
            pltpu.VMEM((GROUP_ROWS, EMB), jnp.float32),
            pltpu.SemaphoreType.DMA,
            pltpu.SemaphoreType.DMA,
        ],
        compiler_params=pltpu.CompilerParams(use_tc_tiling_on_sc=False),
    )
    def k(idx_hbm, table_hbm, out_hbm, idx_v, buf0, buf1, sem0, sem1):
        wid = lax.axis_index("s") * NC + lax.axis_index("c")
        pltpu.sync_copy(idx_hbm.at[wid], idx_v)
        base = wid * b_per_w

        def fire(g, buf, sem):
            for b in range(NACC):
                pltpu.async_copy(table_hbm.at[idx_v.at[g * NACC + b]],
                                 buf.at[pl.ds(b * CHUNK, CHUNK)], sem)

        def drain_write(g, buf, sem):
            for b in range(NACC):
                pltpu.make_async_copy(table_hbm.at[idx_v.at[g * NACC + b]],
                                      buf.at[pl.ds(b * CHUNK, CHUNK)], sem).wait()
            pltpu.sync_copy(buf, out_hbm.at[pl.ds(base + g * GROUP_ROWS,
                                                  GROUP_ROWS)])

        fire(0, buf0, sem0)

        def body(g2, carry):
            g0 = 2 * g2
            fire(g0 + 1, buf1, sem1)
            drain_write(g0, buf0, sem0)

            @pl.when(g0 + 2 < n_groups)
            def _():
                fire(g0 + 2, buf0, sem0)

            drain_write(g0 + 1, buf1, sem1)
            return carry

        lax.fori_loop(0, n_groups // 2, body, 0)

    return k


def kernel(y, table):
    batch, n_fields = y.shape
    B = batch * n_fields
    idx = y.astype(jnp.int32).reshape(NW, B // NW // CHUNK, CHUNK)
    out = _make_gather(B, table.shape[0])(idx, table)
    return out.reshape(batch, n_fields, EMB)
